# Initial kernel scaffold; baseline (speedup 1.0000x reference)
import jax
import jax.numpy as jnp
from jax.experimental import pallas as pl


def kernel(aa_indices, attention_mask, emb_table, W_dense, b_dense, W_out, b_out):
    raise NotImplementedError("write your pallas kernel here")


# speedup vs baseline: 49.1607x; 49.1607x over previous
import functools

import jax
import jax.numpy as jnp
from jax import lax
from jax.experimental import pallas as pl
from jax.experimental.pallas import tpu as pltpu
from jax.experimental.pallas import tpu_sc as plsc

_NC = 2
_NS = 16
_NW = _NC * _NS
_L = 16

_VPAD = 128
_OPAD = 128


def _hist_body(S, bpw, idx_hbm, mask_hbm, counts_hbm, idx_v, mask_v, counts_v):
    wid = lax.axis_index("s") * _NC + lax.axis_index("c")
    base = wid * bpw
    pltpu.sync_copy(idx_hbm.at[pl.ds(base, bpw)], idx_v)
    pltpu.sync_copy(mask_hbm.at[pl.ds(base, bpw)], mask_v)

    zeros = jnp.zeros((_L,), jnp.float32)

    def zero_row(i, carry):
        for c in range(_VPAD // _L):
            counts_v[i, pl.ds(c * _L, _L)] = zeros
        return carry

    lax.fori_loop(0, bpw, zero_row, 0)

    lanes = lax.iota(jnp.int32, _L)
    for g in range(bpw // _L):
        rows = g * _L + lanes

        def step(s, carry):
            scol = jnp.full((_L,), s, jnp.int32)
            iv = plsc.load_gather(idx_v, [rows, scol])
            mv = plsc.load_gather(mask_v, [rows, scol])
            plsc.addupdate_scatter(counts_v, [rows, iv], mv.astype(jnp.float32))
            return carry

        lax.fori_loop(0, S, step, 0)

    pltpu.sync_copy(counts_v, counts_hbm.at[pl.ds(base, bpw)])


def _head_body(counts_ref, emb_ref, wd_ref, bd_ref, wo_ref, bo_ref, out_ref):
    c = counts_ref[...]
    denom = jnp.clip(jnp.sum(c, axis=1, keepdims=True), 1e-9, None)
    pooled = lax.dot(c, emb_ref[...], preferred_element_type=jnp.float32) / denom
    h = jnp.tanh(
        lax.dot(pooled, wd_ref[...], preferred_element_type=jnp.float32)
        + bd_ref[...]
    )
    out_ref[...] = (
        lax.dot(h, wo_ref[...], preferred_element_type=jnp.float32) + bo_ref[...]
    )


def kernel(aa_indices, attention_mask, emb_table, W_dense, b_dense, W_out, b_out):
    B, S = aa_indices.shape
    V, D = emb_table.shape
    NL = W_out.shape[1]
    bpw = B // _NW

    mesh = plsc.VectorSubcoreMesh(core_axis_name="c", subcore_axis_name="s")
    hist = pl.kernel(
        functools.partial(_hist_body, S, bpw),
        out_type=jax.ShapeDtypeStruct((B, _VPAD), jnp.float32),
        mesh=mesh,
        scratch_types=[
            pltpu.VMEM((bpw, S), jnp.int32),
            pltpu.VMEM((bpw, S), jnp.int32),
            pltpu.VMEM((bpw, _VPAD), jnp.float32),
        ],
        compiler_params=pltpu.CompilerParams(needs_layout_passes=False),
    )
    counts = hist(aa_indices, attention_mask)

    emb_pad = jnp.zeros((_VPAD, D), jnp.float32).at[:V].set(emb_table)
    wo_pad = jnp.zeros((D, _OPAD), jnp.float32).at[:, :NL].set(W_out)
    bo_pad = jnp.zeros((1, _OPAD), jnp.float32).at[0, :NL].set(b_out)
    bd = b_dense.reshape(1, D)

    BM = 256
    out_pad = pl.pallas_call(
        _head_body,
        grid=(B // BM,),
        in_specs=[
            pl.BlockSpec((BM, _VPAD), lambda i: (i, 0)),
            pl.BlockSpec((_VPAD, D), lambda i: (0, 0)),
            pl.BlockSpec((D, D), lambda i: (0, 0)),
            pl.BlockSpec((1, D), lambda i: (0, 0)),
            pl.BlockSpec((D, _OPAD), lambda i: (0, 0)),
            pl.BlockSpec((1, _OPAD), lambda i: (0, 0)),
        ],
        out_specs=pl.BlockSpec((BM, _OPAD), lambda i: (i, 0)),
        out_shape=jax.ShapeDtypeStruct((B, _OPAD), jnp.float32),
    )(counts, emb_pad, W_dense, bd, wo_pad, bo_pad)

    return out_pad[:, :NL]
